# baseline (speedup 1.0000x reference)
import functools

import jax
import jax.numpy as jnp
from jax import lax
from jax.experimental import pallas as pl
from jax.experimental.pallas import tpu as pltpu
from jax.experimental.pallas import tpu_sc as plsc

N = 10000
E = 320000
F = 128
H = 256

NC, NS = 2, 16
CHUNK = 128
T_CH = 80
C_TOT = T_CH * NC * NS
E_PAD = C_TOT * CHUNK
N_PAD = 10240
SLICE = N_PAD // NS
ROWS = 2000

_mesh = plsc.VectorSubcoreMesh(
    core_axis_name="c", subcore_axis_name="s", num_cores=NC, num_subcores=NS)


@functools.partial(
    pl.kernel,
    out_type=jax.ShapeDtypeStruct((NC, NS, SLICE), jnp.float32),
    mesh=_mesh,
    scratch_types=[
        pltpu.VMEM((2 * T_CH, CHUNK), jnp.int32),
        pltpu.VMEM((1, CHUNK), jnp.float32),
        pltpu.VMEM_SHARED((N_PAD,), jnp.float32),
        pltpu.SemaphoreType.DMA,
    ],
)
def _sc_degree(edge_hbm, ones_hbm, zeros_hbm, out_hbm, e_v, ones_v, deg_sh, dsem):
    c = lax.axis_index("c")
    s = lax.axis_index("s")
    base = c * (NS * T_CH) + s * T_CH
    pltpu.sync_copy(edge_hbm.at[pl.ds(2 * base, 2 * T_CH)], e_v)
    pltpu.sync_copy(ones_hbm, ones_v)
    pltpu.sync_copy(zeros_hbm.at[pl.ds(s * SLICE, SLICE)],
                    deg_sh.at[pl.ds(s * SLICE, SLICE)])
    plsc.subcore_barrier()

    def fire(j, carry):
        pltpu.async_copy(ones_v.at[0], deg_sh.at[e_v.at[2 * j + 1]], dsem,
                         add=True)
        return carry

    def drain(j, carry):
        pltpu.make_async_copy(ones_v.at[0], deg_sh.at[e_v.at[2 * j + 1]],
                              dsem).wait()
        return carry

    lax.fori_loop(0, T_CH, fire, 0)
    lax.fori_loop(0, T_CH, drain, 0)
    plsc.subcore_barrier()
    pltpu.sync_copy(deg_sh.at[pl.ds(s * SLICE, SLICE)], out_hbm.at[c, s])


@functools.partial(
    pl.kernel,
    out_type=jax.ShapeDtypeStruct((NC, NS, SLICE, F), jnp.float32),
    mesh=_mesh,
    scratch_types=[
        pltpu.VMEM((2, CHUNK), jnp.int32),
        pltpu.VMEM((2, CHUNK), jnp.int32),
        pltpu.VMEM((2, CHUNK), jnp.int32),
        pltpu.VMEM((2, CHUNK), jnp.int32),
        pltpu.VMEM((CHUNK, F), jnp.float32),
        pltpu.VMEM((CHUNK, F), jnp.float32),
        pltpu.VMEM_SHARED((N_PAD, F), jnp.float32),
        pltpu.SemaphoreType.DMA,
        pltpu.SemaphoreType.DMA,
        pltpu.SemaphoreType.DMA,
        pltpu.SemaphoreType.DMA,
        pltpu.SemaphoreType.DMA,
        pltpu.SemaphoreType.DMA,
        pltpu.SemaphoreType.DMA,
        pltpu.SemaphoreType.DMA,
    ],
)
def _sc_aggregate(edge_hbm, z_hbm, zeros_hbm, out_hbm,
                  ib0, ib1, ib2, ib3, msg0, msg1, acc_sh,
                  isem0, isem1, isem2, isem3, gsem0, gsem1, ssem0, ssem1):
    c = lax.axis_index("c")
    s = lax.axis_index("s")
    base = c * (NS * T_CH) + s * T_CH
    ib = (ib0, ib1, ib2, ib3)
    isem = (isem0, isem1, isem2, isem3)
    msgs = (msg0, msg1)
    gsem = (gsem0, gsem1)
    ssem = (ssem0, ssem1)

    def i_start(j, bi):
        pltpu.async_copy(edge_hbm.at[pl.ds(2 * (base + j), 2)], ib[bi],
                         isem[bi])

    def i_wait(j, bi):
        pltpu.make_async_copy(edge_hbm.at[pl.ds(2 * (base + j), 2)], ib[bi],
                              isem[bi]).wait()

    def g_start(bi, bm):
        pltpu.async_copy(z_hbm.at[ib[bi].at[0]], msgs[bm], gsem[bm])

    def g_wait(bi, bm):
        pltpu.make_async_copy(z_hbm.at[ib[bi].at[0]], msgs[bm],
                              gsem[bm]).wait()

    def s_start(bi, bm):
        pltpu.async_copy(msgs[bm], acc_sh.at[ib[bi].at[1]], ssem[bm],
                         add=True)

    def s_wait(bi, bm):
        pltpu.make_async_copy(msgs[bm], acc_sh.at[ib[bi].at[1]],
                              ssem[bm]).wait()

    i_start(0, 0)
    i_start(1, 1)
    pltpu.sync_copy(zeros_hbm.at[pl.ds(s * SLICE, SLICE)],
                    acc_sh.at[pl.ds(s * SLICE, SLICE)])
    i_wait(0, 0)
    g_start(0, 0)
    plsc.subcore_barrier()

    g_wait(0, 0)
    s_start(0, 0)
    i_start(2, 2)
    i_wait(1, 1)
    g_start(1, 1)

    g_wait(1, 1)
    s_start(1, 1)
    s_wait(0, 0)
    i_start(3, 3)
    i_wait(2, 2)
    g_start(2, 0)

    def body(i, carry):
        for k in range(4):
            j = 2 + 4 * i + k
            bi = (2 + k) % 4
            bm = k % 2
            g_wait(bi, bm)
            s_start(bi, bm)
            s_wait((bi - 1) % 4, 1 - bm)
            i_start(j + 2, k)
            i_wait(j + 1, (bi + 1) % 4)
            g_start((bi + 1) % 4, 1 - bm)
        return carry

    lax.fori_loop(0, (T_CH - 4) // 4, body, 0)

    g_wait(2, 0)
    s_start(2, 0)
    s_wait(1, 1)
    i_wait(T_CH - 1, 3)
    g_start(3, 1)

    g_wait(3, 1)
    s_start(3, 1)
    s_wait(2, 0)
    s_wait(3, 1)

    plsc.subcore_barrier()
    pltpu.sync_copy(acc_sh.at[pl.ds(s * SLICE, SLICE)], out_hbm.at[c, s])


def _tc_pre_body(d0, d1, x, dis_ref, y0_ref):
    dis = lax.rsqrt(d0[...] + d1[...] + 1.0)
    dis_ref[...] = dis
    y0_ref[...] = x[...] * dis


def _tc_mid_body(a0, a1, y0, dis, w1, b1, w2, y1_ref):
    out1 = (a0[...] + a1[...] + y0[...]) * dis[...]
    h = jnp.dot(out1, w1[...], preferred_element_type=jnp.float32) + b1[...]
    h = jnp.maximum(h, 0.0)
    y1_ref[...] = jnp.dot(h, w2[...], preferred_element_type=jnp.float32) * dis[...]


def _tc_fin_body(a0, a1, y1, dis, b2, out_ref):
    out_ref[...] = (a0[...] + a1[...] + y1[...]) * dis[...] + b2[...]


def _row_spec(cols):
    return pl.BlockSpec((ROWS, cols), lambda i: (i, 0))


def _full_spec(r, c):
    return pl.BlockSpec((r, c), lambda i: (0, 0))


_GRID = (N // ROWS,)

_tc_pre = pl.pallas_call(
    _tc_pre_body,
    grid=_GRID,
    in_specs=[_row_spec(1), _row_spec(1), _row_spec(F)],
    out_specs=[_row_spec(1), _row_spec(F)],
    out_shape=[jax.ShapeDtypeStruct((N, 1), jnp.float32),
               jax.ShapeDtypeStruct((N, F), jnp.float32)],
)

_tc_mid = pl.pallas_call(
    _tc_mid_body,
    grid=_GRID,
    in_specs=[_row_spec(F), _row_spec(F), _row_spec(F), _row_spec(1),
              _full_spec(F, H), _full_spec(1, H), _full_spec(H, F)],
    out_specs=_row_spec(F),
    out_shape=jax.ShapeDtypeStruct((N, F), jnp.float32),
)

_tc_fin = pl.pallas_call(
    _tc_fin_body,
    grid=_GRID,
    in_specs=[_row_spec(F), _row_spec(F), _row_spec(F), _row_spec(1),
              _full_spec(1, F)],
    out_specs=_row_spec(F),
    out_shape=jax.ShapeDtypeStruct((N, F), jnp.float32),
)


def kernel(x, edge_index, W1, b1, W2, b2):
    ei = edge_index.astype(jnp.int32)
    n_dummy = E_PAD - E
    pad_dst = N + (jnp.arange(n_dummy, dtype=jnp.int32) % (N_PAD - N))
    pad_src = jnp.full((n_dummy,), N, jnp.int32)
    src = jnp.concatenate([ei[0], pad_src]).reshape(C_TOT, CHUNK)
    dst = jnp.concatenate([ei[1], pad_dst]).reshape(C_TOT, CHUNK)
    edges = jnp.stack([src, dst], axis=1).reshape(2 * C_TOT, CHUNK)

    ones = jnp.ones((1, CHUNK), jnp.float32)
    zeros1 = jnp.zeros((N_PAD,), jnp.float32)
    zeros2 = jnp.zeros((N_PAD, F), jnp.float32)

    deg_parts = _sc_degree(edges, ones, zeros1).reshape(NC, N_PAD)
    d0 = deg_parts[0, :N].reshape(N, 1)
    d1 = deg_parts[1, :N].reshape(N, 1)

    dis, y0 = _tc_pre(d0, d1, x)

    y0_pad = jnp.zeros((N_PAD, F), jnp.float32).at[:N].set(y0)
    acc = _sc_aggregate(edges, y0_pad, zeros2).reshape(NC, N_PAD, F)

    y1 = _tc_mid(acc[0, :N], acc[1, :N], y0, dis,
                 W1, b1.reshape(1, H), W2)

    y1_pad = jnp.zeros((N_PAD, F), jnp.float32).at[:N].set(y1)
    acc2 = _sc_aggregate(edges, y1_pad, zeros2).reshape(NC, N_PAD, F)

    out = _tc_fin(acc2[0, :N], acc2[1, :N], y1, dis, b2.reshape(1, F))
    return out


# speedup vs baseline: 2.9275x; 2.9275x over previous
import functools

import jax
import jax.numpy as jnp
from jax import lax
from jax.experimental import pallas as pl
from jax.experimental.pallas import tpu as pltpu
from jax.experimental.pallas import tpu_sc as plsc

N = 10000
E = 320000
F = 128
H = 256

NC, NS = 2, 16
CHUNK = 128
T_CH = 80
C_TOT = T_CH * NC * NS
E_PAD = C_TOT * CHUNK
N_PAD = 10240
SLICE = N_PAD // NS
ROWS = 2000

_mesh = plsc.VectorSubcoreMesh(
    core_axis_name="c", subcore_axis_name="s", num_cores=NC, num_subcores=NS)


@functools.partial(
    pl.kernel,
    out_type=jax.ShapeDtypeStruct((NC, NS, SLICE), jnp.float32),
    mesh=_mesh,
    scratch_types=[
        pltpu.VMEM((2 * T_CH, CHUNK), jnp.int32),
        pltpu.VMEM((1, CHUNK), jnp.float32),
        pltpu.VMEM_SHARED((N_PAD,), jnp.float32),
        pltpu.SemaphoreType.DMA,
    ],
)
def _sc_degree(edge_hbm, ones_hbm, zeros_hbm, out_hbm, e_v, ones_v, deg_sh, dsem):
    c = lax.axis_index("c")
    s = lax.axis_index("s")
    base = c * (NS * T_CH) + s * T_CH
    pltpu.sync_copy(edge_hbm.at[pl.ds(2 * base, 2 * T_CH)], e_v)
    pltpu.sync_copy(ones_hbm, ones_v)
    pltpu.sync_copy(zeros_hbm.at[pl.ds(s * SLICE, SLICE)],
                    deg_sh.at[pl.ds(s * SLICE, SLICE)])
    plsc.subcore_barrier()

    def fire(j, carry):
        pltpu.async_copy(ones_v.at[0], deg_sh.at[e_v.at[2 * j + 1]], dsem,
                         add=True)
        return carry

    def drain(j, carry):
        pltpu.make_async_copy(ones_v.at[0], deg_sh.at[e_v.at[2 * j + 1]],
                              dsem).wait()
        return carry

    lax.fori_loop(0, T_CH, fire, 0)
    lax.fori_loop(0, T_CH, drain, 0)
    plsc.subcore_barrier()
    pltpu.sync_copy(deg_sh.at[pl.ds(s * SLICE, SLICE)], out_hbm.at[c, s])


@functools.partial(
    pl.kernel,
    out_type=jax.ShapeDtypeStruct((NC, NS, SLICE, F), jnp.float32),
    mesh=_mesh,
    scratch_types=[
        pltpu.VMEM((2, CHUNK), jnp.int32),
        pltpu.VMEM((2, CHUNK), jnp.int32),
        pltpu.VMEM((2, CHUNK), jnp.int32),
        pltpu.VMEM((2, CHUNK), jnp.int32),
        pltpu.VMEM((CHUNK, F), jnp.float32),
        pltpu.VMEM((CHUNK, F), jnp.float32),
        pltpu.VMEM_SHARED((N_PAD, F), jnp.float32),
        pltpu.SemaphoreType.DMA,
        pltpu.SemaphoreType.DMA,
        pltpu.SemaphoreType.DMA,
        pltpu.SemaphoreType.DMA,
        pltpu.SemaphoreType.DMA,
        pltpu.SemaphoreType.DMA,
        pltpu.SemaphoreType.DMA,
        pltpu.SemaphoreType.DMA,
    ],
)
def _sc_aggregate(edge_hbm, z_hbm, zeros_hbm, out_hbm,
                  ib0, ib1, ib2, ib3, msg0, msg1, acc_sh,
                  isem0, isem1, isem2, isem3, gsem0, gsem1, ssem0, ssem1):
    c = lax.axis_index("c")
    s = lax.axis_index("s")
    wid = c * NS + s
    ib = (ib0, ib1, ib2, ib3)
    isem = (isem0, isem1, isem2, isem3)
    msgs = (msg0, msg1)
    gsem = (gsem0, gsem1)
    ssem = (ssem0, ssem1)

    def i_start(j, bi):
        pltpu.async_copy(edge_hbm.at[pl.ds(2 * (wid + NC * NS * j), 2)],
                         ib[bi], isem[bi])

    def i_wait(j, bi):
        pltpu.make_async_copy(edge_hbm.at[pl.ds(2 * (wid + NC * NS * j), 2)],
                              ib[bi], isem[bi]).wait()

    def g_start(bi, bm):
        pltpu.async_copy(z_hbm.at[ib[bi].at[0]], msgs[bm], gsem[bm])

    def g_wait(bi, bm):
        pltpu.make_async_copy(z_hbm.at[ib[bi].at[0]], msgs[bm],
                              gsem[bm]).wait()

    def s_start(bi, bm):
        pltpu.async_copy(msgs[bm], acc_sh.at[ib[bi].at[1]], ssem[bm],
                         add=True)

    def s_wait(bi, bm):
        pltpu.make_async_copy(msgs[bm], acc_sh.at[ib[bi].at[1]],
                              ssem[bm]).wait()

    i_start(0, 0)
    i_start(1, 1)
    pltpu.sync_copy(zeros_hbm.at[pl.ds(s * SLICE, SLICE)],
                    acc_sh.at[pl.ds(s * SLICE, SLICE)])
    i_wait(0, 0)
    g_start(0, 0)
    plsc.subcore_barrier()

    g_wait(0, 0)
    s_start(0, 0)
    i_start(2, 2)
    i_wait(1, 1)
    g_start(1, 1)

    g_wait(1, 1)
    s_start(1, 1)
    s_wait(0, 0)
    i_start(3, 3)
    i_wait(2, 2)
    g_start(2, 0)

    def body(i, carry):
        for k in range(4):
            j = 2 + 4 * i + k
            bi = (2 + k) % 4
            bm = k % 2
            g_wait(bi, bm)
            s_start(bi, bm)
            s_wait((bi - 1) % 4, 1 - bm)
            i_start(j + 2, k)
            i_wait(j + 1, (bi + 1) % 4)
            g_start((bi + 1) % 4, 1 - bm)
        return carry

    lax.fori_loop(0, (T_CH - 4) // 4, body, 0)

    g_wait(2, 0)
    s_start(2, 0)
    s_wait(1, 1)
    i_wait(T_CH - 1, 3)
    g_start(3, 1)

    g_wait(3, 1)
    s_start(3, 1)
    s_wait(2, 0)
    s_wait(3, 1)

    plsc.subcore_barrier()
    pltpu.sync_copy(acc_sh.at[pl.ds(s * SLICE, SLICE)], out_hbm.at[c, s])


def _tc_pre_body(d0, d1, x, dis_ref, y0_ref):
    dis = lax.rsqrt(d0[...] + d1[...] + 1.0)
    dis_ref[...] = dis
    y0_ref[...] = x[...] * dis


def _tc_mid_body(a0, a1, y0, dis, w1, b1, w2, y1_ref):
    out1 = (a0[...] + a1[...] + y0[...]) * dis[...]
    h = jnp.dot(out1, w1[...], preferred_element_type=jnp.float32) + b1[...]
    h = jnp.maximum(h, 0.0)
    y1_ref[...] = jnp.dot(h, w2[...], preferred_element_type=jnp.float32) * dis[...]


def _tc_fin_body(a0, a1, y1, dis, b2, out_ref):
    out_ref[...] = (a0[...] + a1[...] + y1[...]) * dis[...] + b2[...]


def _row_spec(cols):
    return pl.BlockSpec((ROWS, cols), lambda i: (i, 0))


def _full_spec(r, c):
    return pl.BlockSpec((r, c), lambda i: (0, 0))


_GRID = (N // ROWS,)

_tc_pre = pl.pallas_call(
    _tc_pre_body,
    grid=_GRID,
    in_specs=[_row_spec(1), _row_spec(1), _row_spec(F)],
    out_specs=[_row_spec(1), _row_spec(F)],
    out_shape=[jax.ShapeDtypeStruct((N, 1), jnp.float32),
               jax.ShapeDtypeStruct((N, F), jnp.float32)],
)

_tc_mid = pl.pallas_call(
    _tc_mid_body,
    grid=_GRID,
    in_specs=[_row_spec(F), _row_spec(F), _row_spec(F), _row_spec(1),
              _full_spec(F, H), _full_spec(1, H), _full_spec(H, F)],
    out_specs=_row_spec(F),
    out_shape=jax.ShapeDtypeStruct((N, F), jnp.float32),
)

_tc_fin = pl.pallas_call(
    _tc_fin_body,
    grid=_GRID,
    in_specs=[_row_spec(F), _row_spec(F), _row_spec(F), _row_spec(1),
              _full_spec(1, F)],
    out_specs=_row_spec(F),
    out_shape=jax.ShapeDtypeStruct((N, F), jnp.float32),
)


def kernel(x, edge_index, W1, b1, W2, b2):
    ei = edge_index.astype(jnp.int32)
    n_dummy = E_PAD - E
    pad_dst = N + (jnp.arange(n_dummy, dtype=jnp.int32) % (N_PAD - N))
    pad_src = pad_dst
    src = jnp.concatenate([ei[0], pad_src]).reshape(C_TOT, CHUNK)
    dst = jnp.concatenate([ei[1], pad_dst]).reshape(C_TOT, CHUNK)
    edges = jnp.stack([src, dst], axis=1).reshape(2 * C_TOT, CHUNK)

    ones = jnp.ones((1, CHUNK), jnp.float32)
    zeros1 = jnp.zeros((N_PAD,), jnp.float32)
    zeros2 = jnp.zeros((N_PAD, F), jnp.float32)

    deg_parts = _sc_degree(edges, ones, zeros1).reshape(NC, N_PAD)
    d0 = deg_parts[0, :N].reshape(N, 1)
    d1 = deg_parts[1, :N].reshape(N, 1)

    dis, y0 = _tc_pre(d0, d1, x)

    y0_pad = jnp.zeros((N_PAD, F), jnp.float32).at[:N].set(y0)
    acc = _sc_aggregate(edges, y0_pad, zeros2).reshape(NC, N_PAD, F)

    y1 = _tc_mid(acc[0, :N], acc[1, :N], y0, dis,
                 W1, b1.reshape(1, H), W2)

    y1_pad = jnp.zeros((N_PAD, F), jnp.float32).at[:N].set(y1)
    acc2 = _sc_aggregate(edges, y1_pad, zeros2).reshape(NC, N_PAD, F)

    out = _tc_fin(acc2[0, :N], acc2[1, :N], y1, dis, b2.reshape(1, F))
    return out
